# baseline (speedup 1.0000x reference)
import functools

import jax
import jax.numpy as jnp
from jax import lax
from jax.experimental import pallas as pl
from jax.experimental.pallas import tpu as pltpu
from jax.experimental.pallas import tpu_sc as plsc

_TEMPERATURE = 0.1
_EPS = 1e-12
_TILE = 4096


def _scan_body(nt, tile, b, x_ref, bank_ref, idx_ref, xn_scr, max_scr, arg_scr):
    i = pl.program_id(0)

    @pl.when(i == 0)
    def _init():
        x = x_ref[...]
        n = jnp.sqrt(jnp.sum(x * x, axis=1, keepdims=True))
        xn_scr[...] = x / jnp.maximum(n, _EPS)
        max_scr[...] = jnp.full((b,), -jnp.inf, jnp.float32)
        arg_scr[...] = jnp.zeros((b,), jnp.int32)

    bt = bank_ref[...]
    s = jnp.sum(bt * bt, axis=1, keepdims=True)
    btn = bt * jnp.where(s > 1e-24, lax.rsqrt(s), 0.0)
    sim = lax.dot_general(
        btn, xn_scr[...], (((1,), (1,)), ((), ())),
        preferred_element_type=jnp.float32)
    m = jnp.max(sim, axis=0)
    rows = lax.broadcasted_iota(jnp.int32, sim.shape, 0)
    amax = jnp.min(jnp.where(sim == m[None, :], rows, tile), axis=0)
    better = m > max_scr[...]
    arg_scr[...] = jnp.where(better, i * tile + amax, arg_scr[...])
    max_scr[...] = jnp.where(better, m, max_scr[...])

    @pl.when(i == nt - 1)
    def _fin():
        idx_ref[...] = arg_scr[...]


def _argmax_scan(out0, bank):
    b, d = out0.shape
    v = bank.shape[0]
    nt = v // _TILE
    return pl.pallas_call(
        functools.partial(_scan_body, nt, _TILE, b),
        grid=(nt,),
        in_specs=[
            pl.BlockSpec((b, d), lambda i: (0, 0)),
            pl.BlockSpec((_TILE, d), lambda i: (i, 0)),
        ],
        out_specs=pl.BlockSpec((b,), lambda i: (0,)),
        out_shape=jax.ShapeDtypeStruct((b,), jnp.int32),
        scratch_shapes=[
            pltpu.VMEM((b, d), jnp.float32),
            pltpu.VMEM((b,), jnp.float32),
            pltpu.VMEM((b,), jnp.int32),
        ],
        compiler_params=pltpu.CompilerParams(
            dimension_semantics=("arbitrary",)),
    )(out0, bank)


@functools.lru_cache(maxsize=None)
def _build_sc_gather(v, d, b):
    info = plsc.get_sparse_core_info()
    nw = info.num_cores * info.num_subcores
    b_per_w = b // nw
    nc = info.num_cores
    mesh = plsc.VectorSubcoreMesh(core_axis_name="c", subcore_axis_name="s")

    @functools.partial(
        pl.kernel, mesh=mesh,
        out_type=jax.ShapeDtypeStruct((b, d), jnp.float32),
        scratch_types=[
            pltpu.VMEM((b_per_w,), jnp.int32),
            pltpu.VMEM((b_per_w, d), jnp.float32),
            pltpu.SemaphoreType.DMA,
        ],
        compiler_params=pltpu.CompilerParams(use_tc_tiling_on_sc=False),
    )
    def gather(table_hbm, idx_hbm, out_hbm, idx_v, rows_v, sem):
        wid = lax.axis_index("s") * nc + lax.axis_index("c")
        base = wid * b_per_w
        pltpu.sync_copy(idx_hbm.at[pl.ds(base, b_per_w)], idx_v)
        pltpu.async_copy(table_hbm.at[idx_v], rows_v, sem).wait()
        pltpu.sync_copy(rows_v, out_hbm.at[pl.ds(base, b_per_w)])

    return gather


def _loss_body(b, a_ref, c_ref, out_ref):
    a = a_ref[...]
    c = c_ref[...]
    za = a / jnp.maximum(jnp.sqrt(jnp.sum(a * a, axis=1, keepdims=True)), _EPS)
    zc = c / jnp.maximum(jnp.sqrt(jnp.sum(c * c, axis=1, keepdims=True)), _EPS)
    logits = lax.dot_general(
        za, zc, (((1,), (1,)), ((), ())),
        preferred_element_type=jnp.float32) / _TEMPERATURE
    m0 = jnp.max(logits, axis=1, keepdims=True)
    lse0 = jnp.log(jnp.sum(jnp.exp(logits - m0), axis=1)) + m0[:, 0]
    m1 = jnp.max(logits, axis=0, keepdims=True)
    lse1 = jnp.log(jnp.sum(jnp.exp(logits - m1), axis=0)) + m1[0, :]
    r = lax.broadcasted_iota(jnp.int32, logits.shape, 0)
    col = lax.broadcasted_iota(jnp.int32, logits.shape, 1)
    diag = jnp.sum(jnp.where(r == col, logits, 0.0))
    loss = (0.5 * (jnp.sum(lse0) + jnp.sum(lse1)) - diag) / b
    out_ref[...] = loss[None, None]


def _ntxent(nn0, out1):
    b, d = nn0.shape
    res = pl.pallas_call(
        functools.partial(_loss_body, b),
        out_shape=jax.ShapeDtypeStruct((1, 1), jnp.float32),
    )(nn0, out1)
    return res[0, 0]


def kernel(out0, out1, bank):
    b, d = out0.shape
    v = bank.shape[0]
    idx = _argmax_scan(out0, bank)
    nn0 = _build_sc_gather(v, d, b)(bank, idx)
    return _ntxent(nn0, out1)


# speedup vs baseline: 1.0307x; 1.0307x over previous
import functools

import jax
import jax.numpy as jnp
from jax import lax
from jax.experimental import pallas as pl
from jax.experimental.pallas import tpu as pltpu
from jax.experimental.pallas import tpu_sc as plsc

_TEMPERATURE = 0.1
_EPS = 1e-12
_TILE = 4096


def _scan_body(nt, tile, b, x_ref, bank_ref, idx_ref, xn_scr, max_scr, arg_scr):
    i = pl.program_id(0)

    @pl.when(i == 0)
    def _init():
        x = x_ref[...]
        n = jnp.sqrt(jnp.sum(x * x, axis=1, keepdims=True))
        xn_scr[...] = x / jnp.maximum(n, _EPS)
        max_scr[...] = jnp.full((b,), jnp.iinfo(jnp.int32).min, jnp.int32)
        arg_scr[...] = jnp.zeros((b,), jnp.int32)

    bt = bank_ref[...]
    s = jnp.sum(bt * bt, axis=1, keepdims=True)
    btn = bt * jnp.where(s > 1e-24, lax.rsqrt(s), 0.0)
    sim = lax.dot_general(
        btn, xn_scr[...], (((1,), (1,)), ((), ())),
        preferred_element_type=jnp.float32)
    k = lax.bitcast_convert_type(sim, jnp.int32)
    k = k ^ (lax.shift_right_arithmetic(k, 31) & jnp.int32(0x7FFFFFFF))
    rows = lax.broadcasted_iota(jnp.int32, sim.shape, 0)
    key = (k & jnp.int32(-tile)) | (rows ^ jnp.int32(tile - 1))
    kmax = jnp.max(key, axis=0)
    simpart = kmax & jnp.int32(-tile)
    better = simpart > max_scr[...]
    amax = (kmax & jnp.int32(tile - 1)) ^ jnp.int32(tile - 1)
    arg_scr[...] = jnp.where(better, i * tile + amax, arg_scr[...])
    max_scr[...] = jnp.where(better, simpart, max_scr[...])

    @pl.when(i == nt - 1)
    def _fin():
        idx_ref[...] = arg_scr[...]


def _argmax_scan(out0, bank):
    b, d = out0.shape
    v = bank.shape[0]
    nt = v // _TILE
    return pl.pallas_call(
        functools.partial(_scan_body, nt, _TILE, b),
        grid=(nt,),
        in_specs=[
            pl.BlockSpec((b, d), lambda i: (0, 0)),
            pl.BlockSpec((_TILE, d), lambda i: (i, 0)),
        ],
        out_specs=pl.BlockSpec((b,), lambda i: (0,)),
        out_shape=jax.ShapeDtypeStruct((b,), jnp.int32),
        scratch_shapes=[
            pltpu.VMEM((b, d), jnp.float32),
            pltpu.VMEM((b,), jnp.int32),
            pltpu.VMEM((b,), jnp.int32),
        ],
        compiler_params=pltpu.CompilerParams(
            dimension_semantics=("arbitrary",)),
    )(out0, bank)


@functools.lru_cache(maxsize=None)
def _build_sc_gather(v, d, b):
    info = plsc.get_sparse_core_info()
    nw = info.num_cores * info.num_subcores
    b_per_w = b // nw
    nc = info.num_cores
    mesh = plsc.VectorSubcoreMesh(core_axis_name="c", subcore_axis_name="s")

    @functools.partial(
        pl.kernel, mesh=mesh,
        out_type=jax.ShapeDtypeStruct((b, d), jnp.float32),
        scratch_types=[
            pltpu.VMEM((b_per_w,), jnp.int32),
            pltpu.VMEM((b_per_w, d), jnp.float32),
            pltpu.SemaphoreType.DMA,
        ],
        compiler_params=pltpu.CompilerParams(use_tc_tiling_on_sc=False),
    )
    def gather(table_hbm, idx_hbm, out_hbm, idx_v, rows_v, sem):
        wid = lax.axis_index("s") * nc + lax.axis_index("c")
        base = wid * b_per_w
        pltpu.sync_copy(idx_hbm.at[pl.ds(base, b_per_w)], idx_v)
        pltpu.async_copy(table_hbm.at[idx_v], rows_v, sem).wait()
        pltpu.sync_copy(rows_v, out_hbm.at[pl.ds(base, b_per_w)])

    return gather


def _loss_body(b, a_ref, c_ref, out_ref):
    a = a_ref[...]
    c = c_ref[...]
    za = a / jnp.maximum(jnp.sqrt(jnp.sum(a * a, axis=1, keepdims=True)), _EPS)
    zc = c / jnp.maximum(jnp.sqrt(jnp.sum(c * c, axis=1, keepdims=True)), _EPS)
    logits = lax.dot_general(
        za, zc, (((1,), (1,)), ((), ())),
        preferred_element_type=jnp.float32) / _TEMPERATURE
    m0 = jnp.max(logits, axis=1, keepdims=True)
    lse0 = jnp.log(jnp.sum(jnp.exp(logits - m0), axis=1)) + m0[:, 0]
    m1 = jnp.max(logits, axis=0, keepdims=True)
    lse1 = jnp.log(jnp.sum(jnp.exp(logits - m1), axis=0)) + m1[0, :]
    r = lax.broadcasted_iota(jnp.int32, logits.shape, 0)
    col = lax.broadcasted_iota(jnp.int32, logits.shape, 1)
    diag = jnp.sum(jnp.where(r == col, logits, 0.0))
    loss = (0.5 * (jnp.sum(lse0) + jnp.sum(lse1)) - diag) / b
    out_ref[...] = loss[None, None]


def _ntxent(nn0, out1):
    b, d = nn0.shape
    res = pl.pallas_call(
        functools.partial(_loss_body, b),
        out_shape=jax.ShapeDtypeStruct((1, 1), jnp.float32),
    )(nn0, out1)
    return res[0, 0]


def kernel(out0, out1, bank):
    b, d = out0.shape
    v = bank.shape[0]
    idx = _argmax_scan(out0, bank)
    nn0 = _build_sc_gather(v, d, b)(bank, idx)
    return _ntxent(nn0, out1)


# speedup vs baseline: 1.2796x; 1.2415x over previous
import functools

import jax
import jax.numpy as jnp
from jax import lax
from jax.experimental import pallas as pl
from jax.experimental.pallas import tpu as pltpu
from jax.experimental.pallas import tpu_sc as plsc

_TEMPERATURE = 0.1
_EPS = 1e-12
_TILE = 4096


def _scan_body(nt, tile, b, x_ref, bank_ref, idx_ref, xn_scr, max_scr, arg_scr):
    i = pl.program_id(0)

    @pl.when(i == 0)
    def _init():
        x = x_ref[...]
        n = jnp.sqrt(jnp.sum(x * x, axis=1, keepdims=True))
        xn_scr[...] = x / jnp.maximum(n, _EPS)
        max_scr[...] = jnp.full((b,), -jnp.inf, jnp.float32)
        arg_scr[...] = jnp.zeros((b,), jnp.int32)

    bt = bank_ref[...]
    s = jnp.sum(bt * bt, axis=1, keepdims=True)
    btn = bt * jnp.where(s > 1e-24, lax.rsqrt(s), 0.0)
    sim = lax.dot_general(
        btn, xn_scr[...], (((1,), (1,)), ((), ())),
        preferred_element_type=jnp.float32)
    bits = lax.bitcast_convert_type(sim, jnp.int32)
    rows = lax.broadcasted_iota(jnp.int32, sim.shape, 0)
    key = (bits & jnp.int32(-tile)) | (rows ^ jnp.int32(tile - 1))
    keyf = lax.bitcast_convert_type(key, jnp.float32)
    kmax = lax.bitcast_convert_type(jnp.max(keyf, axis=0), jnp.int32)
    simpart = lax.bitcast_convert_type(kmax & jnp.int32(-tile), jnp.float32)
    better = simpart > max_scr[...]
    amax = (kmax & jnp.int32(tile - 1)) ^ jnp.int32(tile - 1)
    arg_scr[...] = jnp.where(better, i * tile + amax, arg_scr[...])
    max_scr[...] = jnp.where(better, simpart, max_scr[...])

    @pl.when(i == nt - 1)
    def _fin():
        idx_ref[...] = arg_scr[...]


def _argmax_scan(out0, bank):
    b, d = out0.shape
    v = bank.shape[0]
    nt = v // _TILE
    return pl.pallas_call(
        functools.partial(_scan_body, nt, _TILE, b),
        grid=(nt,),
        in_specs=[
            pl.BlockSpec((b, d), lambda i: (0, 0)),
            pl.BlockSpec((_TILE, d), lambda i: (i, 0)),
        ],
        out_specs=pl.BlockSpec((b,), lambda i: (0,)),
        out_shape=jax.ShapeDtypeStruct((b,), jnp.int32),
        scratch_shapes=[
            pltpu.VMEM((b, d), jnp.float32),
            pltpu.VMEM((b,), jnp.float32),
            pltpu.VMEM((b,), jnp.int32),
        ],
        compiler_params=pltpu.CompilerParams(
            dimension_semantics=("arbitrary",)),
    )(out0, bank)


@functools.lru_cache(maxsize=None)
def _build_sc_gather(v, d, b):
    info = plsc.get_sparse_core_info()
    nw = info.num_cores * info.num_subcores
    b_per_w = b // nw
    nc = info.num_cores
    mesh = plsc.VectorSubcoreMesh(core_axis_name="c", subcore_axis_name="s")

    @functools.partial(
        pl.kernel, mesh=mesh,
        out_type=jax.ShapeDtypeStruct((b, d), jnp.float32),
        scratch_types=[
            pltpu.VMEM((b_per_w,), jnp.int32),
            pltpu.VMEM((b_per_w, d), jnp.float32),
            pltpu.SemaphoreType.DMA,
        ],
        compiler_params=pltpu.CompilerParams(use_tc_tiling_on_sc=False),
    )
    def gather(table_hbm, idx_hbm, out_hbm, idx_v, rows_v, sem):
        wid = lax.axis_index("s") * nc + lax.axis_index("c")
        base = wid * b_per_w
        pltpu.sync_copy(idx_hbm.at[pl.ds(base, b_per_w)], idx_v)
        pltpu.async_copy(table_hbm.at[idx_v], rows_v, sem).wait()
        pltpu.sync_copy(rows_v, out_hbm.at[pl.ds(base, b_per_w)])

    return gather


def _loss_body(b, a_ref, c_ref, out_ref):
    a = a_ref[...]
    c = c_ref[...]
    za = a / jnp.maximum(jnp.sqrt(jnp.sum(a * a, axis=1, keepdims=True)), _EPS)
    zc = c / jnp.maximum(jnp.sqrt(jnp.sum(c * c, axis=1, keepdims=True)), _EPS)
    logits = lax.dot_general(
        za, zc, (((1,), (1,)), ((), ())),
        preferred_element_type=jnp.float32) / _TEMPERATURE
    m0 = jnp.max(logits, axis=1, keepdims=True)
    lse0 = jnp.log(jnp.sum(jnp.exp(logits - m0), axis=1)) + m0[:, 0]
    m1 = jnp.max(logits, axis=0, keepdims=True)
    lse1 = jnp.log(jnp.sum(jnp.exp(logits - m1), axis=0)) + m1[0, :]
    r = lax.broadcasted_iota(jnp.int32, logits.shape, 0)
    col = lax.broadcasted_iota(jnp.int32, logits.shape, 1)
    diag = jnp.sum(jnp.where(r == col, logits, 0.0))
    loss = (0.5 * (jnp.sum(lse0) + jnp.sum(lse1)) - diag) / b
    out_ref[...] = loss[None, None]


def _ntxent(nn0, out1):
    b, d = nn0.shape
    res = pl.pallas_call(
        functools.partial(_loss_body, b),
        out_shape=jax.ShapeDtypeStruct((1, 1), jnp.float32),
    )(nn0, out1)
    return res[0, 0]


def kernel(out0, out1, bank):
    b, d = out0.shape
    v = bank.shape[0]
    idx = _argmax_scan(out0, bank)
    nn0 = _build_sc_gather(v, d, b)(bank, idx)
    return _ntxent(nn0, out1)


# speedup vs baseline: 1.6543x; 1.2928x over previous
import functools

import jax
import jax.numpy as jnp
from jax import lax
from jax.experimental import pallas as pl
from jax.experimental.pallas import tpu as pltpu
from jax.experimental.pallas import tpu_sc as plsc

_TEMPERATURE = 0.1
_EPS = 1e-12
_TILE = 4096


def _scan_body(nt, tile, b, x_ref, bank_ref, idx_ref, pairs_ref,
               xn_scr, max_scr, arg_scr):
    i = pl.program_id(0)

    @pl.when(i == 0)
    def _init():
        x = x_ref[...]
        n = jnp.sqrt(jnp.sum(x * x, axis=1, keepdims=True))
        xn_scr[...] = x / jnp.maximum(n, _EPS)
        max_scr[...] = jnp.full((b,), -jnp.inf, jnp.float32)
        arg_scr[...] = jnp.zeros((b,), jnp.int32)

    bt = bank_ref[...]
    pairs_ref[...] = jnp.concatenate([bt, bt], axis=1)
    s = jnp.sum(bt * bt, axis=1, keepdims=True)
    btn = bt * jnp.where(s > 1e-24, lax.rsqrt(s), 0.0)
    sim = lax.dot_general(
        btn, xn_scr[...], (((1,), (1,)), ((), ())),
        preferred_element_type=jnp.float32)
    bits = lax.bitcast_convert_type(sim, jnp.int32)
    rows = lax.broadcasted_iota(jnp.int32, sim.shape, 0)
    key = (bits & jnp.int32(-tile)) | (rows ^ jnp.int32(tile - 1))
    keyf = lax.bitcast_convert_type(key, jnp.float32)
    kmax = lax.bitcast_convert_type(jnp.max(keyf, axis=0), jnp.int32)
    simpart = lax.bitcast_convert_type(kmax & jnp.int32(-tile), jnp.float32)
    better = simpart > max_scr[...]
    amax = (kmax & jnp.int32(tile - 1)) ^ jnp.int32(tile - 1)
    arg_scr[...] = jnp.where(better, i * tile + amax, arg_scr[...])
    max_scr[...] = jnp.where(better, simpart, max_scr[...])

    @pl.when(i == nt - 1)
    def _fin():
        idx_ref[...] = arg_scr[...]


def _argmax_scan(out0, bank):
    b, d = out0.shape
    v = bank.shape[0]
    nt = v // _TILE
    return pl.pallas_call(
        functools.partial(_scan_body, nt, _TILE, b),
        grid=(nt,),
        in_specs=[
            pl.BlockSpec((b, d), lambda i: (0, 0)),
            pl.BlockSpec((_TILE, d), lambda i: (i, 0)),
        ],
        out_specs=[
            pl.BlockSpec((b,), lambda i: (0,)),
            pl.BlockSpec((_TILE // 2, 2 * d), lambda i: (i, 0)),
        ],
        out_shape=[
            jax.ShapeDtypeStruct((b,), jnp.int32),
            jax.ShapeDtypeStruct((v // 2, 2 * d), jnp.float32),
        ],
        scratch_shapes=[
            pltpu.VMEM((b, d), jnp.float32),
            pltpu.VMEM((b,), jnp.float32),
            pltpu.VMEM((b,), jnp.int32),
        ],
        compiler_params=pltpu.CompilerParams(
            dimension_semantics=("arbitrary",)),
    )(out0, bank)


@functools.lru_cache(maxsize=None)
def _build_sc_gather(v2, d2, b):
    info = plsc.get_sparse_core_info()
    nw = info.num_cores * info.num_subcores
    nl = info.num_lanes
    b_per_w = b // nw
    nc = info.num_cores
    mesh = plsc.VectorSubcoreMesh(core_axis_name="c", subcore_axis_name="s")

    @functools.partial(
        pl.kernel, mesh=mesh,
        out_type=jax.ShapeDtypeStruct((b, d2), jnp.float32),
        scratch_types=[
            pltpu.VMEM((b_per_w,), jnp.int32),
            pltpu.VMEM((b_per_w, d2), jnp.float32),
            pltpu.SemaphoreType.DMA,
        ],
    )
    def gather(table_hbm, idx_hbm, out_hbm, idx_v, rows_v, sem):
        wid = lax.axis_index("s") * nc + lax.axis_index("c")
        base = wid * b_per_w
        pltpu.sync_copy(idx_hbm.at[pl.ds(base, b_per_w)], idx_v)
        pltpu.async_copy(table_hbm.at[idx_v], rows_v, sem).wait()
        pltpu.sync_copy(rows_v, out_hbm.at[pl.ds(base, b_per_w)])

    return gather


def _loss_body(b, d, wide_ref, c_ref, out_ref):
    a = wide_ref[...][:, :d]
    c = c_ref[...]
    za = a / jnp.maximum(jnp.sqrt(jnp.sum(a * a, axis=1, keepdims=True)), _EPS)
    zc = c / jnp.maximum(jnp.sqrt(jnp.sum(c * c, axis=1, keepdims=True)), _EPS)
    logits = lax.dot_general(
        za, zc, (((1,), (1,)), ((), ())),
        preferred_element_type=jnp.float32) / _TEMPERATURE
    m0 = jnp.max(logits, axis=1, keepdims=True)
    lse0 = jnp.log(jnp.sum(jnp.exp(logits - m0), axis=1)) + m0[:, 0]
    m1 = jnp.max(logits, axis=0, keepdims=True)
    lse1 = jnp.log(jnp.sum(jnp.exp(logits - m1), axis=0)) + m1[0, :]
    r = lax.broadcasted_iota(jnp.int32, logits.shape, 0)
    col = lax.broadcasted_iota(jnp.int32, logits.shape, 1)
    diag = jnp.sum(jnp.where(r == col, logits, 0.0))
    loss = (0.5 * (jnp.sum(lse0) + jnp.sum(lse1)) - diag) / b
    out_ref[...] = loss[None, None]


def _ntxent(wide, out1):
    b, d = out1.shape
    res = pl.pallas_call(
        functools.partial(_loss_body, b, d),
        out_shape=jax.ShapeDtypeStruct((1, 1), jnp.float32),
    )(wide, out1)
    return res[0, 0]


def kernel(out0, out1, bank):
    b, d = out0.shape
    v = bank.shape[0]
    idx, pairs = _argmax_scan(out0, bank)
    wide = _build_sc_gather(v, 2 * d, b)(pairs, idx)
    return _ntxent(wide, out1)


# speedup vs baseline: 1.7073x; 1.0320x over previous
import functools

import jax
import jax.numpy as jnp
from jax import lax
from jax.experimental import pallas as pl
from jax.experimental.pallas import tpu as pltpu
from jax.experimental.pallas import tpu_sc as plsc

_TEMPERATURE = 0.1
_EPS = 1e-12
_TILE = 4096


def _scan_body(nt, tile, b, x_ref, bankt_ref, idx_ref, table_ref,
               xn_scr, max_scr, arg_scr):
    i = pl.program_id(0)

    @pl.when(i == 0)
    def _init():
        x = x_ref[...]
        n = jnp.sqrt(jnp.sum(x * x, axis=1, keepdims=True))
        xn_scr[...] = x / jnp.maximum(n, _EPS)
        max_scr[...] = jnp.full((b,), -jnp.inf, jnp.float32)
        arg_scr[...] = jnp.zeros((b,), jnp.int32)

    bt = bankt_ref[...]
    btr = jnp.transpose(bt, (1, 0))
    table_ref[...] = jnp.concatenate([btr, btr], axis=1)
    s = jnp.sum(bt * bt, axis=0, keepdims=True)
    btn = bt * jnp.where(s > 1e-24, lax.rsqrt(s), 0.0)
    sim = lax.dot_general(
        xn_scr[...], btn, (((1,), (0,)), ((), ())),
        preferred_element_type=jnp.float32)
    bits = lax.bitcast_convert_type(sim, jnp.int32)
    rows = lax.broadcasted_iota(jnp.int32, sim.shape, 1)
    key = (bits & jnp.int32(-tile)) | (rows ^ jnp.int32(tile - 1))
    keyf = lax.bitcast_convert_type(key, jnp.float32)
    kmax = lax.bitcast_convert_type(jnp.max(keyf, axis=1), jnp.int32)
    simpart = lax.bitcast_convert_type(kmax & jnp.int32(-tile), jnp.float32)
    better = simpart > max_scr[...]
    amax = (kmax & jnp.int32(tile - 1)) ^ jnp.int32(tile - 1)
    arg_scr[...] = jnp.where(better, i * tile + amax, arg_scr[...])
    max_scr[...] = jnp.where(better, simpart, max_scr[...])

    @pl.when(i == nt - 1)
    def _fin():
        idx_ref[...] = arg_scr[...]


def _argmax_scan(out0, bankt):
    b, d = out0.shape
    v = bankt.shape[1]
    nt = v // _TILE
    return pl.pallas_call(
        functools.partial(_scan_body, nt, _TILE, b),
        grid=(nt,),
        in_specs=[
            pl.BlockSpec((b, d), lambda i: (0, 0)),
            pl.BlockSpec((d, _TILE), lambda i: (0, i)),
        ],
        out_specs=[
            pl.BlockSpec((b,), lambda i: (0,)),
            pl.BlockSpec((_TILE, 2 * d), lambda i: (i, 0)),
        ],
        out_shape=[
            jax.ShapeDtypeStruct((b,), jnp.int32),
            jax.ShapeDtypeStruct((v, 2 * d), jnp.float32),
        ],
        scratch_shapes=[
            pltpu.VMEM((b, d), jnp.float32),
            pltpu.VMEM((b,), jnp.float32),
            pltpu.VMEM((b,), jnp.int32),
        ],
        compiler_params=pltpu.CompilerParams(
            dimension_semantics=("arbitrary",)),
    )(out0, bankt)


@functools.lru_cache(maxsize=None)
def _build_sc_gather(v, d2, b):
    info = plsc.get_sparse_core_info()
    nw = info.num_cores * info.num_subcores
    b_per_w = b // nw
    nc = info.num_cores
    mesh = plsc.VectorSubcoreMesh(core_axis_name="c", subcore_axis_name="s")

    @functools.partial(
        pl.kernel, mesh=mesh,
        out_type=jax.ShapeDtypeStruct((b, d2), jnp.float32),
        scratch_types=[
            pltpu.VMEM((b_per_w,), jnp.int32),
            pltpu.VMEM((b_per_w, d2), jnp.float32),
            pltpu.SemaphoreType.DMA,
        ],
    )
    def gather(table_hbm, idx_hbm, out_hbm, idx_v, rows_v, sem):
        wid = lax.axis_index("s") * nc + lax.axis_index("c")
        base = wid * b_per_w
        pltpu.sync_copy(idx_hbm.at[pl.ds(base, b_per_w)], idx_v)
        pltpu.async_copy(table_hbm.at[idx_v], rows_v, sem).wait()
        pltpu.sync_copy(rows_v, out_hbm.at[pl.ds(base, b_per_w)])

    return gather


def _loss_body(b, d, wide_ref, c_ref, out_ref):
    a = wide_ref[...][:, :d]
    c = c_ref[...]
    za = a / jnp.maximum(jnp.sqrt(jnp.sum(a * a, axis=1, keepdims=True)), _EPS)
    zc = c / jnp.maximum(jnp.sqrt(jnp.sum(c * c, axis=1, keepdims=True)), _EPS)
    logits = lax.dot_general(
        za, zc, (((1,), (1,)), ((), ())),
        preferred_element_type=jnp.float32) / _TEMPERATURE
    m0 = jnp.max(logits, axis=1, keepdims=True)
    lse0 = jnp.log(jnp.sum(jnp.exp(logits - m0), axis=1)) + m0[:, 0]
    m1 = jnp.max(logits, axis=0, keepdims=True)
    lse1 = jnp.log(jnp.sum(jnp.exp(logits - m1), axis=0)) + m1[0, :]
    r = lax.broadcasted_iota(jnp.int32, logits.shape, 0)
    col = lax.broadcasted_iota(jnp.int32, logits.shape, 1)
    diag = jnp.sum(jnp.where(r == col, logits, 0.0))
    loss = (0.5 * (jnp.sum(lse0) + jnp.sum(lse1)) - diag) / b
    out_ref[...] = loss[None, None]


def _ntxent(wide, out1):
    b, d = out1.shape
    res = pl.pallas_call(
        functools.partial(_loss_body, b, d),
        out_shape=jax.ShapeDtypeStruct((1, 1), jnp.float32),
    )(wide, out1)
    return res[0, 0]


def kernel(out0, out1, bank):
    b, d = out0.shape
    v = bank.shape[0]
    idx, table = _argmax_scan(out0, bank.T)
    wide = _build_sc_gather(v, 2 * d, b)(table, idx)
    return _ntxent(wide, out1)


# speedup vs baseline: 2.0844x; 1.2209x over previous
import functools

import jax
import jax.numpy as jnp
from jax import lax
from jax.experimental import pallas as pl
from jax.experimental.pallas import tpu as pltpu
from jax.experimental.pallas import tpu_sc as plsc

_TEMPERATURE = 0.1
_EPS = 1e-12
_TILE = 4096


def _scan_body(nt, tile, b, x_ref, bankt_ref, idx_ref, table_ref,
               xn_scr, max_scr, arg_scr):
    i = pl.program_id(0)

    @pl.when(i == 0)
    def _init():
        x = x_ref[...]
        n = jnp.sqrt(jnp.sum(x * x, axis=1, keepdims=True))
        xn_scr[...] = x / jnp.maximum(n, _EPS)
        max_scr[...] = jnp.full((b,), -jnp.inf, jnp.float32)
        arg_scr[...] = jnp.zeros((b,), jnp.int32)

    bt = bankt_ref[...]
    d = bt.shape[0]
    eye2 = jnp.where(
        lax.broadcasted_iota(jnp.int32, (d, 2 * d), 0)
        == (lax.broadcasted_iota(jnp.int32, (d, 2 * d), 1) & (d - 1)),
        1.0, 0.0)
    table_ref[...] = lax.dot_general(
        bt, eye2, (((0,), (0,)), ((), ())),
        preferred_element_type=jnp.float32)
    s = jnp.sum(bt * bt, axis=0, keepdims=True)
    btn = bt * jnp.where(s > 1e-24, lax.rsqrt(s), 0.0)
    sim = lax.dot_general(
        btn, xn_scr[...], (((0,), (1,)), ((), ())),
        preferred_element_type=jnp.float32)
    bits = lax.bitcast_convert_type(sim, jnp.int32)
    rows = lax.broadcasted_iota(jnp.int32, sim.shape, 0)
    key = (bits & jnp.int32(-tile)) | (rows ^ jnp.int32(tile - 1))
    keyf = lax.bitcast_convert_type(key, jnp.float32)
    kmax = lax.bitcast_convert_type(jnp.max(keyf, axis=0), jnp.int32)
    simpart = lax.bitcast_convert_type(kmax & jnp.int32(-tile), jnp.float32)
    better = simpart > max_scr[...]
    amax = (kmax & jnp.int32(tile - 1)) ^ jnp.int32(tile - 1)
    arg_scr[...] = jnp.where(better, i * tile + amax, arg_scr[...])
    max_scr[...] = jnp.where(better, simpart, max_scr[...])

    @pl.when(i == nt - 1)
    def _fin():
        idx_ref[...] = arg_scr[...]


def _argmax_scan(out0, bankt):
    b, d = out0.shape
    v = bankt.shape[1]
    nt = v // _TILE
    return pl.pallas_call(
        functools.partial(_scan_body, nt, _TILE, b),
        grid=(nt,),
        in_specs=[
            pl.BlockSpec((b, d), lambda i: (0, 0)),
            pl.BlockSpec((d, _TILE), lambda i: (0, i)),
        ],
        out_specs=[
            pl.BlockSpec((b,), lambda i: (0,)),
            pl.BlockSpec((_TILE, 2 * d), lambda i: (i, 0)),
        ],
        out_shape=[
            jax.ShapeDtypeStruct((b,), jnp.int32),
            jax.ShapeDtypeStruct((v, 2 * d), jnp.float32),
        ],
        scratch_shapes=[
            pltpu.VMEM((b, d), jnp.float32),
            pltpu.VMEM((b,), jnp.float32),
            pltpu.VMEM((b,), jnp.int32),
        ],
        compiler_params=pltpu.CompilerParams(
            dimension_semantics=("arbitrary",)),
    )(out0, bankt)


@functools.lru_cache(maxsize=None)
def _build_sc_gather(v, d2, b):
    info = plsc.get_sparse_core_info()
    nw = info.num_cores * info.num_subcores
    b_per_w = b // nw
    nc = info.num_cores
    mesh = plsc.VectorSubcoreMesh(core_axis_name="c", subcore_axis_name="s")

    @functools.partial(
        pl.kernel, mesh=mesh,
        out_type=jax.ShapeDtypeStruct((b, d2), jnp.float32),
        scratch_types=[
            pltpu.VMEM((b_per_w,), jnp.int32),
            pltpu.VMEM((b_per_w, d2), jnp.float32),
            pltpu.SemaphoreType.DMA,
        ],
    )
    def gather(table_hbm, idx_hbm, out_hbm, idx_v, rows_v, sem):
        wid = lax.axis_index("s") * nc + lax.axis_index("c")
        base = wid * b_per_w
        pltpu.sync_copy(idx_hbm.at[pl.ds(base, b_per_w)], idx_v)
        pltpu.async_copy(table_hbm.at[idx_v], rows_v, sem).wait()
        pltpu.sync_copy(rows_v, out_hbm.at[pl.ds(base, b_per_w)])

    return gather


def _loss_body(b, d, wide_ref, c_ref, out_ref):
    a = wide_ref[...][:, :d]
    c = c_ref[...]
    za = a / jnp.maximum(jnp.sqrt(jnp.sum(a * a, axis=1, keepdims=True)), _EPS)
    zc = c / jnp.maximum(jnp.sqrt(jnp.sum(c * c, axis=1, keepdims=True)), _EPS)
    logits = lax.dot_general(
        za, zc, (((1,), (1,)), ((), ())),
        preferred_element_type=jnp.float32) / _TEMPERATURE
    m0 = jnp.max(logits, axis=1, keepdims=True)
    lse0 = jnp.log(jnp.sum(jnp.exp(logits - m0), axis=1)) + m0[:, 0]
    m1 = jnp.max(logits, axis=0, keepdims=True)
    lse1 = jnp.log(jnp.sum(jnp.exp(logits - m1), axis=0)) + m1[0, :]
    r = lax.broadcasted_iota(jnp.int32, logits.shape, 0)
    col = lax.broadcasted_iota(jnp.int32, logits.shape, 1)
    diag = jnp.sum(jnp.where(r == col, logits, 0.0))
    loss = (0.5 * (jnp.sum(lse0) + jnp.sum(lse1)) - diag) / b
    out_ref[...] = loss[None, None]


def _ntxent(wide, out1):
    b, d = out1.shape
    res = pl.pallas_call(
        functools.partial(_loss_body, b, d),
        out_shape=jax.ShapeDtypeStruct((1, 1), jnp.float32),
    )(wide, out1)
    return res[0, 0]


def kernel(out0, out1, bank):
    b, d = out0.shape
    v = bank.shape[0]
    idx, table = _argmax_scan(out0, bank.T)
    wide = _build_sc_gather(v, 2 * d, b)(table, idx)
    return _ntxent(wide, out1)


# speedup vs baseline: 2.1584x; 1.0355x over previous
import functools

import jax
import jax.numpy as jnp
from jax import lax
from jax.experimental import pallas as pl
from jax.experimental.pallas import tpu as pltpu
from jax.experimental.pallas import tpu_sc as plsc

_TEMPERATURE = 0.1
_EPS = 1e-12
_TILE = 4096


def _scan_body(nt, tile, b, xt_ref, bankt_ref, idx_ref, table_ref,
               xn_scr, max_scr, arg_scr):
    i = pl.program_id(0)

    @pl.when(i == 0)
    def _init():
        xt = xt_ref[...]
        n = jnp.sum(xt * xt, axis=0, keepdims=True)
        xn = xt * jnp.where(n > 1e-24, lax.rsqrt(n), 0.0)
        xn_scr[...] = xn.astype(jnp.bfloat16)
        max_scr[...] = jnp.full((b,), -jnp.inf, jnp.float32)
        arg_scr[...] = jnp.zeros((b,), jnp.int32)

    bt = bankt_ref[...]
    d = bt.shape[0]
    eye2 = jnp.where(
        lax.broadcasted_iota(jnp.int32, (d, 2 * d), 0)
        == (lax.broadcasted_iota(jnp.int32, (d, 2 * d), 1) & (d - 1)),
        1.0, 0.0)
    table_ref[...] = lax.dot_general(
        bt, eye2, (((0,), (0,)), ((), ())),
        preferred_element_type=jnp.float32)
    s = jnp.sum(bt * bt, axis=0, keepdims=True)
    btn = (bt * jnp.where(s > 1e-24, lax.rsqrt(s), 0.0)).astype(jnp.bfloat16)
    sim = lax.dot_general(
        btn, xn_scr[...], (((0,), (0,)), ((), ())),
        preferred_element_type=jnp.float32)
    bits = lax.bitcast_convert_type(sim, jnp.int32)
    rows = lax.broadcasted_iota(jnp.int32, sim.shape, 0)
    key = (bits & jnp.int32(-tile)) | (rows ^ jnp.int32(tile - 1))
    keyf = lax.bitcast_convert_type(key, jnp.float32)
    kmax = lax.bitcast_convert_type(jnp.max(keyf, axis=0), jnp.int32)
    simpart = lax.bitcast_convert_type(kmax & jnp.int32(-tile), jnp.float32)
    better = simpart > max_scr[...]
    amax = (kmax & jnp.int32(tile - 1)) ^ jnp.int32(tile - 1)
    arg_scr[...] = jnp.where(better, i * tile + amax, arg_scr[...])
    max_scr[...] = jnp.where(better, simpart, max_scr[...])

    @pl.when(i == nt - 1)
    def _fin():
        idx_ref[...] = arg_scr[...]


def _argmax_scan(out0t, bankt):
    d, b = out0t.shape
    v = bankt.shape[1]
    nt = v // _TILE
    return pl.pallas_call(
        functools.partial(_scan_body, nt, _TILE, b),
        grid=(nt,),
        in_specs=[
            pl.BlockSpec((d, b), lambda i: (0, 0)),
            pl.BlockSpec((d, _TILE), lambda i: (0, i)),
        ],
        out_specs=[
            pl.BlockSpec((b,), lambda i: (0,)),
            pl.BlockSpec((_TILE, 2 * d), lambda i: (i, 0)),
        ],
        out_shape=[
            jax.ShapeDtypeStruct((b,), jnp.int32),
            jax.ShapeDtypeStruct((v, 2 * d), jnp.float32),
        ],
        scratch_shapes=[
            pltpu.VMEM((d, b), jnp.bfloat16),
            pltpu.VMEM((b,), jnp.float32),
            pltpu.VMEM((b,), jnp.int32),
        ],
        compiler_params=pltpu.CompilerParams(
            dimension_semantics=("arbitrary",)),
    )(out0t, bankt)


@functools.lru_cache(maxsize=None)
def _build_sc_gather(v, d2, b):
    info = plsc.get_sparse_core_info()
    nw = info.num_cores * info.num_subcores
    b_per_w = b // nw
    nc = info.num_cores
    mesh = plsc.VectorSubcoreMesh(core_axis_name="c", subcore_axis_name="s")

    @functools.partial(
        pl.kernel, mesh=mesh,
        out_type=jax.ShapeDtypeStruct((b, d2), jnp.float32),
        scratch_types=[
            pltpu.VMEM((b_per_w,), jnp.int32),
            pltpu.VMEM((b_per_w, d2), jnp.float32),
            pltpu.SemaphoreType.DMA,
        ],
    )
    def gather(table_hbm, idx_hbm, out_hbm, idx_v, rows_v, sem):
        wid = lax.axis_index("s") * nc + lax.axis_index("c")
        base = wid * b_per_w
        pltpu.sync_copy(idx_hbm.at[pl.ds(base, b_per_w)], idx_v)
        pltpu.async_copy(table_hbm.at[idx_v], rows_v, sem).wait()
        pltpu.sync_copy(rows_v, out_hbm.at[pl.ds(base, b_per_w)])

    return gather


def _loss_body(b, d, wide_ref, ct_ref, out_ref):
    a = wide_ref[...][:, :d]
    ct = ct_ref[...]
    za = a / jnp.maximum(jnp.sqrt(jnp.sum(a * a, axis=1, keepdims=True)), _EPS)
    n1 = jnp.sum(ct * ct, axis=0, keepdims=True)
    zct = ct * jnp.where(n1 > 1e-24, lax.rsqrt(n1), 0.0)
    logits = lax.dot_general(
        za, zct, (((1,), (0,)), ((), ())),
        preferred_element_type=jnp.float32) / _TEMPERATURE
    m0 = jnp.max(logits, axis=1, keepdims=True)
    lse0 = jnp.log(jnp.sum(jnp.exp(logits - m0), axis=1)) + m0[:, 0]
    m1 = jnp.max(logits, axis=0, keepdims=True)
    lse1 = jnp.log(jnp.sum(jnp.exp(logits - m1), axis=0)) + m1[0, :]
    r = lax.broadcasted_iota(jnp.int32, logits.shape, 0)
    col = lax.broadcasted_iota(jnp.int32, logits.shape, 1)
    diag = jnp.sum(jnp.where(r == col, logits, 0.0))
    loss = (0.5 * (jnp.sum(lse0) + jnp.sum(lse1)) - diag) / b
    out_ref[...] = loss[None, None]


def _ntxent(wide, out1t):
    d, b = out1t.shape
    res = pl.pallas_call(
        functools.partial(_loss_body, b, d),
        out_shape=jax.ShapeDtypeStruct((1, 1), jnp.float32),
    )(wide, out1t)
    return res[0, 0]


def kernel(out0, out1, bank):
    b, d = out0.shape
    v = bank.shape[0]
    idx, table = _argmax_scan(out0.T, bank.T)
    wide = _build_sc_gather(v, 2 * d, b)(table, idx)
    return _ntxent(wide, out1.T)
